# baseline (speedup 1.0000x reference)
import functools

import jax
import jax.numpy as jnp
from jax import lax
from jax.experimental import pallas as pl
from jax.experimental.pallas import tpu as pltpu
from jax.experimental.pallas import tpu_sc as plsc

N_NODES = 10000
N_EDGES = 320000
D_FEAT = 128

NC = 2
NS = 16
NW = NC * NS
EPW = N_EDGES // NW
CHUNK = 100
CPW = EPW // CHUNK
NHALF = 2
CPH = CPW // NHALF
N_PAD = 10240
RPS = N_PAD // NS

_MESH = plsc.VectorSubcoreMesh(core_axis_name="c", subcore_axis_name="s")


@functools.partial(
    pl.kernel,
    out_type=jax.ShapeDtypeStruct((NC, N_PAD, D_FEAT), jnp.float32),
    mesh=_MESH,
    scratch_types=[
        pltpu.VMEM((CPH, CHUNK), jnp.int32),
        pltpu.VMEM((CPH, CHUNK), jnp.int32),
        pltpu.VMEM((2, CHUNK, D_FEAT), jnp.float32),
        pltpu.VMEM_SHARED((N_PAD, D_FEAT), jnp.float32),
        pltpu.SemaphoreType.DMA,
        pltpu.SemaphoreType.DMA,
        pltpu.SemaphoreType.DMA,
    ],
)
def _scatter_gather(x_hbm, src_hbm, dst_hbm, z_hbm, out_hbm,
                    src_v, dst_v, rows_v, acc, gsem, s0, s1):
    c = lax.axis_index("c")
    s = lax.axis_index("s")
    wid = c * NS + s

    pltpu.sync_copy(z_hbm, acc.at[pl.ds(s * RPS, RPS)])
    plsc.subcore_barrier()

    rows0 = rows_v.at[0]
    rows1 = rows_v.at[1]

    def wait_gather(buf):
        pltpu.make_async_copy(x_hbm.at[src_v.at[0]], buf, gsem).wait()

    def wait_scatter(buf, ssem):
        pltpu.make_async_copy(buf, acc.at[dst_v.at[0]], ssem).wait()

    for h in range(NHALF):
        pltpu.sync_copy(src_hbm.at[wid, h], src_v)
        pltpu.sync_copy(dst_hbm.at[wid, h], dst_v)

        pltpu.async_copy(x_hbm.at[src_v.at[0]], rows0, gsem)

        def step(i, carry):
            j0 = 2 * i
            j1 = j0 + 1
            wait_gather(rows0)

            @pl.when(i > 0)
            def _():
                wait_scatter(rows1, s1)

            pltpu.async_copy(x_hbm.at[src_v.at[j1]], rows1, gsem)
            pltpu.async_copy(rows0, acc.at[dst_v.at[j0]], s0, priority=1, add=True)
            wait_gather(rows1)

            @pl.when(i + 1 < CPH // 2)
            def _():
                wait_scatter(rows0, s0)
                pltpu.async_copy(x_hbm.at[src_v.at[j0 + 2]], rows0, gsem)

            pltpu.async_copy(rows1, acc.at[dst_v.at[j1]], s1, priority=1, add=True)
            return carry

        lax.fori_loop(0, CPH // 2, step, 0)
        wait_scatter(rows0, s0)
        wait_scatter(rows1, s1)

    plsc.subcore_barrier()
    pltpu.sync_copy(acc.at[pl.ds(s * RPS, RPS)],
                    out_hbm.at[c, pl.ds(s * RPS, RPS)])


def _combine_body(p_ref, o_ref):
    o_ref[...] = p_ref[0] + p_ref[1]


def _combine(partials):
    rows = N_NODES // 10
    return pl.pallas_call(
        _combine_body,
        grid=(10,),
        in_specs=[pl.BlockSpec((NC, rows, D_FEAT), lambda i: (0, i, 0))],
        out_specs=pl.BlockSpec((rows, D_FEAT), lambda i: (i, 0)),
        out_shape=jax.ShapeDtypeStruct((N_NODES, D_FEAT), jnp.float32),
    )(partials)


def kernel(x, edge_index):
    src = edge_index[0].reshape(NW, NHALF, CPH, CHUNK)
    dst = edge_index[1].reshape(NW, NHALF, CPH, CHUNK)
    zeros = jnp.zeros((RPS, D_FEAT), jnp.float32)
    partials = _scatter_gather(x, src, dst, zeros)
    return _combine(partials)


# speedup vs baseline: 1.0447x; 1.0447x over previous
import functools

import jax
import jax.numpy as jnp
from jax import lax
from jax.experimental import pallas as pl
from jax.experimental.pallas import tpu as pltpu
from jax.experimental.pallas import tpu_sc as plsc

N_NODES = 10000
N_EDGES = 320000
D_FEAT = 128

NC = 2
NS = 16
NW = NC * NS
EPW = N_EDGES // NW
CHUNK = 125
CPW = EPW // CHUNK
NHALF = 4
CPH = CPW // NHALF
N_PAD = 10240
RPS = N_PAD // NS

_MESH = plsc.VectorSubcoreMesh(core_axis_name="c", subcore_axis_name="s")


@functools.partial(
    pl.kernel,
    out_type=jax.ShapeDtypeStruct((NC, N_PAD, D_FEAT), jnp.float32),
    mesh=_MESH,
    scratch_types=[
        pltpu.VMEM((CPH, CHUNK), jnp.int32),
        pltpu.VMEM((CPH, CHUNK), jnp.int32),
        pltpu.VMEM((2, CHUNK, D_FEAT), jnp.float32),
        pltpu.VMEM_SHARED((N_PAD, D_FEAT), jnp.float32),
        pltpu.SemaphoreType.DMA,
        pltpu.SemaphoreType.DMA,
        pltpu.SemaphoreType.DMA,
    ],
)
def _scatter_gather(x_hbm, src_hbm, dst_hbm, z_hbm, out_hbm,
                    src_v, dst_v, rows_v, acc, gsem, s0, s1):
    c = lax.axis_index("c")
    s = lax.axis_index("s")
    wid = c * NS + s

    pltpu.async_copy(z_hbm, acc.at[pl.ds(s * RPS, RPS)], s0)

    rows0 = rows_v.at[0]
    rows1 = rows_v.at[1]

    def wait_gather(buf):
        pltpu.make_async_copy(x_hbm.at[src_v.at[0]], buf, gsem).wait()

    def wait_scatter(buf, ssem):
        pltpu.make_async_copy(buf, acc.at[dst_v.at[0]], ssem).wait()

    for h in range(NHALF):
        pltpu.sync_copy(src_hbm.at[wid, h], src_v)
        pltpu.sync_copy(dst_hbm.at[wid, h], dst_v)

        pltpu.async_copy(x_hbm.at[src_v.at[0]], rows0, gsem)

        if h == 0:
            pltpu.make_async_copy(z_hbm, acc.at[pl.ds(s * RPS, RPS)],
                                  s0).wait()
            plsc.subcore_barrier()

        def step(i, carry):
            j0 = 2 * i
            j1 = j0 + 1
            wait_gather(rows0)

            @pl.when(i > 0)
            def _():
                wait_scatter(rows1, s1)

            pltpu.async_copy(x_hbm.at[src_v.at[j1]], rows1, gsem)
            pltpu.async_copy(rows0, acc.at[dst_v.at[j0]], s0, priority=1, add=True)
            wait_gather(rows1)

            @pl.when(i + 1 < CPH // 2)
            def _():
                wait_scatter(rows0, s0)
                pltpu.async_copy(x_hbm.at[src_v.at[j0 + 2]], rows0, gsem)

            pltpu.async_copy(rows1, acc.at[dst_v.at[j1]], s1, priority=1, add=True)
            return carry

        lax.fori_loop(0, CPH // 2, step, 0)
        wait_scatter(rows0, s0)
        wait_scatter(rows1, s1)

    plsc.subcore_barrier()
    pltpu.sync_copy(acc.at[pl.ds(s * RPS, RPS)],
                    out_hbm.at[c, pl.ds(s * RPS, RPS)])


def _combine_body(p_ref, o_ref):
    o_ref[...] = p_ref[0] + p_ref[1]


def _combine(partials):
    rows = N_NODES // 10
    return pl.pallas_call(
        _combine_body,
        grid=(10,),
        in_specs=[pl.BlockSpec((NC, rows, D_FEAT), lambda i: (0, i, 0))],
        out_specs=pl.BlockSpec((rows, D_FEAT), lambda i: (i, 0)),
        out_shape=jax.ShapeDtypeStruct((N_NODES, D_FEAT), jnp.float32),
    )(partials)


def kernel(x, edge_index):
    src = edge_index[0].reshape(NW, NHALF, CPH, CHUNK)
    dst = edge_index[1].reshape(NW, NHALF, CPH, CHUNK)
    zeros = jnp.zeros((RPS, D_FEAT), jnp.float32)
    partials = _scatter_gather(x, src, dst, zeros)
    return _combine(partials)


# speedup vs baseline: 1.0802x; 1.0339x over previous
import functools

import jax
import jax.numpy as jnp
from jax import lax
from jax.experimental import pallas as pl
from jax.experimental.pallas import tpu as pltpu
from jax.experimental.pallas import tpu_sc as plsc

N_NODES = 10000
N_EDGES = 320000
D_FEAT = 128

NC = 2
NS = 16
NW = NC * NS
EPW = N_EDGES // NW
CHUNK = 125
CPW = EPW // CHUNK
NGRP = 8
CPG = CPW // NGRP
N_PAD = 10240
RPS = N_PAD // NS

_MESH = plsc.VectorSubcoreMesh(core_axis_name="c", subcore_axis_name="s")


@functools.partial(
    pl.kernel,
    out_type=jax.ShapeDtypeStruct((NC, N_PAD, D_FEAT), jnp.float32),
    mesh=_MESH,
    scratch_types=[
        pltpu.VMEM((2, CPG, CHUNK), jnp.int32),
        pltpu.VMEM((2, CPG, CHUNK), jnp.int32),
        pltpu.VMEM((2, CHUNK, D_FEAT), jnp.float32),
        pltpu.VMEM_SHARED((N_PAD, D_FEAT), jnp.float32),
        pltpu.SemaphoreType.DMA,
        pltpu.SemaphoreType.DMA,
        pltpu.SemaphoreType.DMA,
        pltpu.SemaphoreType.DMA,
    ],
)
def _scatter_gather(x_hbm, src_hbm, dst_hbm, z_hbm, out_hbm,
                    src_v, dst_v, rows_v, acc, gsem, s0, s1, isem):
    c = lax.axis_index("c")
    s = lax.axis_index("s")
    wid = c * NS + s

    pltpu.async_copy(z_hbm, acc.at[pl.ds(s * RPS, RPS)], s0)

    rows0 = rows_v.at[0]
    rows1 = rows_v.at[1]

    def wait_gather(buf):
        pltpu.make_async_copy(x_hbm.at[src_v.at[0].at[0]], buf, gsem).wait()

    def wait_scatter(buf, ssem):
        pltpu.make_async_copy(buf, acc.at[dst_v.at[0].at[0]], ssem).wait()

    def wait_stage(dst):
        pltpu.make_async_copy(src_hbm.at[wid, 0], dst, isem).wait()

    pltpu.sync_copy(src_hbm.at[wid, 0], src_v.at[0])
    pltpu.sync_copy(dst_hbm.at[wid, 0], dst_v.at[0])
    pltpu.async_copy(x_hbm.at[src_v.at[0].at[0]], rows0, gsem)
    pltpu.make_async_copy(z_hbm, acc.at[pl.ds(s * RPS, RPS)], s0).wait()
    plsc.subcore_barrier()

    for g in range(NGRP):
        p = g % 2
        sp = src_v.at[p]
        dp = dst_v.at[p]
        if g > 0:
            wait_scatter(rows1, s1)
        if g + 1 < NGRP:
            pltpu.async_copy(src_hbm.at[wid, g + 1], src_v.at[1 - p], isem)
            pltpu.async_copy(dst_hbm.at[wid, g + 1], dst_v.at[1 - p], isem)

        def step(i, carry, sp=sp, dp=dp):
            j0 = 2 * i
            j1 = j0 + 1
            wait_gather(rows0)

            @pl.when(i > 0)
            def _():
                wait_scatter(rows1, s1)

            pltpu.async_copy(x_hbm.at[sp.at[j1]], rows1, gsem)
            pltpu.async_copy(rows0, acc.at[dp.at[j0]], s0, priority=1,
                             add=True)
            wait_gather(rows1)

            @pl.when(i + 1 < CPG // 2)
            def _():
                wait_scatter(rows0, s0)
                pltpu.async_copy(x_hbm.at[sp.at[j0 + 2]], rows0, gsem)

            pltpu.async_copy(rows1, acc.at[dp.at[j1]], s1, priority=1,
                             add=True)
            return carry

        lax.fori_loop(0, CPG // 2, step, 0)

        if g + 1 < NGRP:
            wait_stage(src_v.at[1 - p])
            wait_stage(dst_v.at[1 - p])
            wait_scatter(rows0, s0)
            pltpu.async_copy(x_hbm.at[src_v.at[1 - p].at[0]], rows0, gsem)
        else:
            wait_scatter(rows0, s0)
            wait_scatter(rows1, s1)

    plsc.subcore_barrier()
    pltpu.sync_copy(acc.at[pl.ds(s * RPS, RPS)],
                    out_hbm.at[c, pl.ds(s * RPS, RPS)])


def _combine_body(p_ref, o_ref):
    o_ref[...] = p_ref[0] + p_ref[1]


def _combine(partials):
    rows = N_NODES // 10
    return pl.pallas_call(
        _combine_body,
        grid=(10,),
        in_specs=[pl.BlockSpec((NC, rows, D_FEAT), lambda i: (0, i, 0))],
        out_specs=pl.BlockSpec((rows, D_FEAT), lambda i: (i, 0)),
        out_shape=jax.ShapeDtypeStruct((N_NODES, D_FEAT), jnp.float32),
    )(partials)


def kernel(x, edge_index):
    src = edge_index[0].reshape(NW, NGRP, CPG, CHUNK)
    dst = edge_index[1].reshape(NW, NGRP, CPG, CHUNK)
    zeros = jnp.zeros((RPS, D_FEAT), jnp.float32)
    partials = _scatter_gather(x, src, dst, zeros)
    return _combine(partials)


# speedup vs baseline: 1.0851x; 1.0046x over previous
import functools

import jax
import jax.numpy as jnp
from jax import lax
from jax.experimental import pallas as pl
from jax.experimental.pallas import tpu as pltpu
from jax.experimental.pallas import tpu_sc as plsc

N_NODES = 10000
N_EDGES = 320000
D_FEAT = 128

NC = 2
NS = 16
NW = NC * NS
EPW = N_EDGES // NW
CHUNK = 125
CPW = EPW // CHUNK
NGRP = 8
CPG = CPW // NGRP
N_PAD = 10240
RPS = N_PAD // NS

_MESH = plsc.VectorSubcoreMesh(core_axis_name="c", subcore_axis_name="s")


@functools.partial(
    pl.kernel,
    out_type=jax.ShapeDtypeStruct((NC, N_PAD, D_FEAT), jnp.float32),
    mesh=_MESH,
    scratch_types=[
        pltpu.VMEM((2, CPG, CHUNK), jnp.int32),
        pltpu.VMEM((2, CPG, CHUNK), jnp.int32),
        pltpu.VMEM((2, CHUNK, D_FEAT), jnp.float32),
        pltpu.VMEM_SHARED((N_PAD, D_FEAT), jnp.float32),
        pltpu.SemaphoreType.DMA,
        pltpu.SemaphoreType.DMA,
        pltpu.SemaphoreType.DMA,
        pltpu.SemaphoreType.DMA,
    ],
)
def _scatter_gather(x_hbm, src_hbm, dst_hbm, z_hbm, out_hbm,
                    src_v, dst_v, rows_v, acc, gsem, s0, s1, isem):
    c = lax.axis_index("c")
    s = lax.axis_index("s")
    wid = c * NS + s

    pltpu.async_copy(z_hbm, acc.at[pl.ds(s * RPS, RPS)], s0)

    rows0 = rows_v.at[0]
    rows1 = rows_v.at[1]

    def wait_gather(buf):
        pltpu.make_async_copy(x_hbm.at[src_v.at[0].at[0]], buf, gsem).wait()

    def wait_scatter(buf, ssem):
        pltpu.make_async_copy(buf, acc.at[dst_v.at[0].at[0]], ssem).wait()

    def wait_stage(dst):
        pltpu.make_async_copy(src_hbm.at[wid, 0], dst, isem).wait()

    pltpu.sync_copy(src_hbm.at[wid, 0], src_v.at[0])
    pltpu.sync_copy(dst_hbm.at[wid, 0], dst_v.at[0])
    pltpu.async_copy(x_hbm.at[src_v.at[0].at[0]], rows0, gsem)
    pltpu.make_async_copy(z_hbm, acc.at[pl.ds(s * RPS, RPS)], s0).wait()
    plsc.subcore_barrier()

    for g in range(NGRP):
        p = g % 2
        sp = src_v.at[p]
        dp = dst_v.at[p]
        if g > 0:
            wait_scatter(rows1, s1)
        if g + 1 < NGRP:
            pltpu.async_copy(src_hbm.at[wid, g + 1], src_v.at[1 - p], isem)
            pltpu.async_copy(dst_hbm.at[wid, g + 1], dst_v.at[1 - p], isem)

        def step(i, carry, sp=sp, dp=dp):
            j0 = 2 * i
            j1 = j0 + 1
            wait_gather(rows0)

            @pl.when(i > 0)
            def _():
                wait_scatter(rows1, s1)

            pltpu.async_copy(x_hbm.at[sp.at[j1]], rows1, gsem)
            pltpu.async_copy(rows0, acc.at[dp.at[j0]], s0, add=True)
            wait_gather(rows1)

            @pl.when(i + 1 < CPG // 2)
            def _():
                wait_scatter(rows0, s0)
                pltpu.async_copy(x_hbm.at[sp.at[j0 + 2]], rows0, gsem)

            pltpu.async_copy(rows1, acc.at[dp.at[j1]], s1, add=True)
            return carry

        lax.fori_loop(0, CPG // 2, step, 0)

        if g + 1 < NGRP:
            wait_stage(src_v.at[1 - p])
            wait_stage(dst_v.at[1 - p])
            wait_scatter(rows0, s0)
            pltpu.async_copy(x_hbm.at[src_v.at[1 - p].at[0]], rows0, gsem)
        else:
            wait_scatter(rows0, s0)
            wait_scatter(rows1, s1)

    plsc.subcore_barrier()
    pltpu.sync_copy(acc.at[pl.ds(s * RPS, RPS)],
                    out_hbm.at[c, pl.ds(s * RPS, RPS)])


def _combine_body(p_ref, o_ref):
    o_ref[...] = p_ref[0] + p_ref[1]


def _combine(partials):
    rows = N_NODES // 10
    return pl.pallas_call(
        _combine_body,
        grid=(10,),
        in_specs=[pl.BlockSpec((NC, rows, D_FEAT), lambda i: (0, i, 0))],
        out_specs=pl.BlockSpec((rows, D_FEAT), lambda i: (i, 0)),
        out_shape=jax.ShapeDtypeStruct((N_NODES, D_FEAT), jnp.float32),
    )(partials)


def kernel(x, edge_index):
    src = edge_index[0].reshape(NW, NGRP, CPG, CHUNK)
    dst = edge_index[1].reshape(NW, NGRP, CPG, CHUNK)
    zeros = jnp.zeros((RPS, D_FEAT), jnp.float32)
    partials = _scatter_gather(x, src, dst, zeros)
    return _combine(partials)


# speedup vs baseline: 1.1000x; 1.0138x over previous
import functools

import jax
import jax.numpy as jnp
from jax import lax
from jax.experimental import pallas as pl
from jax.experimental.pallas import tpu as pltpu
from jax.experimental.pallas import tpu_sc as plsc

N_NODES = 10000
N_EDGES = 320000
D_FEAT = 128

NC = 2
NS = 16
NW = NC * NS
EPW = N_EDGES // NW
CHUNK = 125
CPW = EPW // CHUNK
NGRP = 8
CPG = CPW // NGRP
N_PAD = 10240
RPS = N_PAD // NS

_MESH = plsc.VectorSubcoreMesh(core_axis_name="c", subcore_axis_name="s")


@functools.partial(
    pl.kernel,
    out_type=jax.ShapeDtypeStruct((NC, N_PAD, D_FEAT), jnp.float32),
    mesh=_MESH,
    scratch_types=[
        pltpu.VMEM((2, CPG, CHUNK), jnp.int32),
        pltpu.VMEM((2, CPG, CHUNK), jnp.int32),
        pltpu.VMEM((2, CHUNK, D_FEAT), jnp.float32),
        pltpu.VMEM_SHARED((N_PAD, D_FEAT), jnp.float32),
        pltpu.SemaphoreType.DMA,
        pltpu.SemaphoreType.DMA,
        pltpu.SemaphoreType.DMA,
        pltpu.SemaphoreType.DMA,
    ],
)
def _scatter_gather(x_hbm, src_hbm, dst_hbm, z_hbm, out_hbm,
                    src_v, dst_v, rows_v, acc, gsem, s0, s1, isem):
    c = lax.axis_index("c")
    s = lax.axis_index("s")
    wid = c * NS + s

    pltpu.async_copy(z_hbm, acc.at[pl.ds(s * RPS, RPS)], s0)

    rows0 = rows_v.at[0]
    rows1 = rows_v.at[1]

    def wait_gather(buf):
        pltpu.make_async_copy(x_hbm.at[src_v.at[0].at[0]], buf, gsem).wait()

    def wait_scatter(buf, ssem):
        pltpu.make_async_copy(buf, acc.at[dst_v.at[0].at[0]], ssem).wait()

    def wait_stage(dst):
        pltpu.make_async_copy(src_hbm.at[wid, 0], dst, isem).wait()

    pltpu.sync_copy(src_hbm.at[wid, 0], src_v.at[0])
    pltpu.sync_copy(dst_hbm.at[wid, 0], dst_v.at[0])
    pltpu.async_copy(x_hbm.at[src_v.at[0].at[0]], rows0, gsem)
    pltpu.make_async_copy(z_hbm, acc.at[pl.ds(s * RPS, RPS)], s0).wait()
    plsc.subcore_barrier()

    for g in range(NGRP):
        p = g % 2
        sp = src_v.at[p]
        dp = dst_v.at[p]
        if g > 0:
            wait_scatter(rows1, s1)
        if g + 1 < NGRP:
            pltpu.async_copy(src_hbm.at[wid, g + 1], src_v.at[1 - p], isem)
            pltpu.async_copy(dst_hbm.at[wid, g + 1], dst_v.at[1 - p], isem)

        def step(i, carry, sp=sp, dp=dp):
            j0 = 2 * i
            j1 = j0 + 1
            wait_gather(rows0)

            @pl.when(i > 0)
            def _():
                wait_scatter(rows1, s1)

            pltpu.async_copy(x_hbm.at[sp.at[j1]], rows1, gsem)
            pltpu.async_copy(rows0, acc.at[dp.at[j0]], s0, add=True)
            wait_gather(rows1)

            @pl.when(i + 1 < CPG // 2)
            def _():
                wait_scatter(rows0, s0)
                pltpu.async_copy(x_hbm.at[sp.at[j0 + 2]], rows0, gsem)

            pltpu.async_copy(rows1, acc.at[dp.at[j1]], s1, add=True)
            return carry

        lax.fori_loop(0, CPG // 2, step, 0)

        if g + 1 < NGRP:
            wait_stage(src_v.at[1 - p])
            wait_stage(dst_v.at[1 - p])
            wait_scatter(rows0, s0)
            pltpu.async_copy(x_hbm.at[src_v.at[1 - p].at[0]], rows0, gsem)
        else:
            wait_scatter(rows0, s0)
            wait_scatter(rows1, s1)

    plsc.subcore_barrier()
    pltpu.sync_copy(acc.at[pl.ds(s * RPS, RPS)],
                    out_hbm.at[c, pl.ds(s * RPS, RPS)])


def _combine_body(p_ref, o_ref):
    o_ref[...] = p_ref[0, :N_NODES] + p_ref[1, :N_NODES]


def _combine(partials):
    return pl.pallas_call(
        _combine_body,
        out_shape=jax.ShapeDtypeStruct((N_NODES, D_FEAT), jnp.float32),
    )(partials)


def kernel(x, edge_index):
    src = edge_index[0].reshape(NW, NGRP, CPG, CHUNK)
    dst = edge_index[1].reshape(NW, NGRP, CPG, CHUNK)
    zeros = jnp.zeros((RPS, D_FEAT), jnp.float32)
    partials = _scatter_gather(x, src, dst, zeros)
    return _combine(partials)


# speedup vs baseline: 1.1106x; 1.0096x over previous
import functools

import jax
import jax.numpy as jnp
from jax import lax
from jax.experimental import pallas as pl
from jax.experimental.pallas import tpu as pltpu
from jax.experimental.pallas import tpu_sc as plsc

N_NODES = 10000
N_EDGES = 320000
D_FEAT = 128

NC = 2
NS = 16
NW = NC * NS
EPW = N_EDGES // NW
CHUNK = 125
CPW = EPW // CHUNK
NGRP = 8
CPG = CPW // NGRP
N_PAD = 10240
RPS = N_PAD // NS

_MESH = plsc.VectorSubcoreMesh(core_axis_name="c", subcore_axis_name="s")


@functools.partial(
    pl.kernel,
    out_type=jax.ShapeDtypeStruct((NC, N_PAD, D_FEAT), jnp.float32),
    mesh=_MESH,
    scratch_types=[
        pltpu.VMEM((2, CPG, CHUNK), jnp.int32),
        pltpu.VMEM((2, CPG, CHUNK), jnp.int32),
        pltpu.VMEM((2, CHUNK, D_FEAT), jnp.float32),
        pltpu.VMEM_SHARED((N_PAD, D_FEAT), jnp.float32),
        pltpu.SemaphoreType.DMA,
        pltpu.SemaphoreType.DMA,
        pltpu.SemaphoreType.DMA,
        pltpu.SemaphoreType.DMA,
    ],
)
def _scatter_gather(x_hbm, src_hbm, dst_hbm, z_hbm, out_hbm,
                    src_v, dst_v, rows_v, acc, gsem, s0, s1, isem):
    c = lax.axis_index("c")
    s = lax.axis_index("s")
    wid = c * NS + s

    pltpu.async_copy(z_hbm, acc.at[pl.ds(s * RPS, RPS)], s0)

    rows0 = rows_v.at[0]
    rows1 = rows_v.at[1]

    def wait_gather(buf):
        pltpu.make_async_copy(x_hbm.at[src_v.at[0].at[0]], buf, gsem).wait()

    def wait_scatter(buf, ssem):
        pltpu.make_async_copy(buf, acc.at[dst_v.at[0].at[0]], ssem).wait()

    def wait_stage(dst):
        pltpu.make_async_copy(src_hbm.at[wid, 0], dst, isem).wait()

    pltpu.sync_copy(src_hbm.at[wid, 0], src_v.at[0])
    pltpu.sync_copy(dst_hbm.at[wid, 0], dst_v.at[0])
    pltpu.async_copy(x_hbm.at[src_v.at[0].at[0]], rows0, gsem)
    pltpu.make_async_copy(z_hbm, acc.at[pl.ds(s * RPS, RPS)], s0).wait()
    plsc.subcore_barrier()

    for g in range(NGRP):
        p = g % 2
        sp = src_v.at[p]
        dp = dst_v.at[p]
        if g > 0:
            wait_scatter(rows1, s1)
        if g + 1 < NGRP:
            pltpu.async_copy(src_hbm.at[wid, g + 1], src_v.at[1 - p], isem)
            pltpu.async_copy(dst_hbm.at[wid, g + 1], dst_v.at[1 - p], isem)

        def step(i, carry, sp=sp, dp=dp):
            j0 = 2 * i
            j1 = j0 + 1
            wait_gather(rows0)

            @pl.when(i > 0)
            def _():
                wait_scatter(rows1, s1)

            pltpu.async_copy(x_hbm.at[sp.at[j1]], rows1, gsem)
            pltpu.async_copy(rows0, acc.at[dp.at[j0]], s0, add=True)
            wait_gather(rows1)

            @pl.when(i + 1 < CPG // 2)
            def _():
                wait_scatter(rows0, s0)
                pltpu.async_copy(x_hbm.at[sp.at[j0 + 2]], rows0, gsem)

            pltpu.async_copy(rows1, acc.at[dp.at[j1]], s1, add=True)
            return carry

        lax.fori_loop(0, CPG // 2, step, 0)

        if g + 1 < NGRP:
            wait_stage(src_v.at[1 - p])
            wait_stage(dst_v.at[1 - p])
            wait_scatter(rows0, s0)
            pltpu.async_copy(x_hbm.at[src_v.at[1 - p].at[0]], rows0, gsem)
        else:
            wait_scatter(rows0, s0)
            wait_scatter(rows1, s1)

    plsc.subcore_barrier()
    pltpu.sync_copy(acc.at[pl.ds(s * RPS, RPS)],
                    out_hbm.at[c, pl.ds(s * RPS, RPS)])


def _combine_body(p_ref, o_ref):
    o_ref[...] = p_ref[0] + p_ref[1]


def _combine(partials):
    rows = N_NODES // 2
    return pl.pallas_call(
        _combine_body,
        grid=(2,),
        in_specs=[pl.BlockSpec((NC, rows, D_FEAT), lambda i: (0, i, 0))],
        out_specs=pl.BlockSpec((rows, D_FEAT), lambda i: (i, 0)),
        out_shape=jax.ShapeDtypeStruct((N_NODES, D_FEAT), jnp.float32),
    )(partials)


def kernel(x, edge_index):
    src = edge_index[0].reshape(NW, NGRP, CPG, CHUNK)
    dst = edge_index[1].reshape(NW, NGRP, CPG, CHUNK)
    zeros = jnp.zeros((RPS, D_FEAT), jnp.float32)
    partials = _scatter_gather(x, src, dst, zeros)
    return _combine(partials)
